# Initial kernel scaffold; baseline (speedup 1.0000x reference)
import jax
import jax.numpy as jnp
from jax.experimental import pallas as pl


def kernel(input_ids, top_k, embed_table, Wq, Wk, Wv, Wo, Wg, Wu, Wd, W_out):
    raise NotImplementedError("write your pallas kernel here")


# speedup vs baseline: 9.3505x; 9.3505x over previous
import functools

import jax
import jax.numpy as jnp
from jax import lax
from jax.experimental import pallas as pl
from jax.experimental.pallas import tpu as pltpu
from jax.experimental.pallas import tpu_sc as plsc

V = 8192
H = 1024
KD = 256
KVH = 4
G = 4
HD = 64
I = 4096
S = 2048
TOPK = 8
L = 2

TB = 256
IB = 512
NI = I // IB
NT = S // TB

TBO = 128
NTO = S // TBO


def _sc_gather(table, ids):
    info = plsc.get_sparse_core_info()
    nw = info.num_cores * info.num_subcores
    b_per_w = S // nw
    mesh = plsc.VectorSubcoreMesh(core_axis_name="c", subcore_axis_name="s")

    @functools.partial(
        pl.kernel,
        mesh=mesh,
        out_type=jax.ShapeDtypeStruct((S, H), jnp.float32),
        scratch_types=[
            pltpu.VMEM((b_per_w,), jnp.int32),
            pltpu.VMEM((b_per_w, H), jnp.float32),
            pltpu.SemaphoreType.DMA,
        ],
    )
    def gather_kernel(table_hbm, idx_hbm, out_hbm, idx_v, rows_v, sem):
        wid = lax.axis_index("s") * info.num_cores + lax.axis_index("c")
        base = wid * b_per_w
        pltpu.sync_copy(idx_hbm.at[pl.ds(base, b_per_w)], idx_v)
        pltpu.async_copy(table_hbm.at[idx_v], rows_v, sem).wait()
        pltpu.sync_copy(rows_v, out_hbm.at[pl.ds(base, b_per_w)])

    return gather_kernel(table, ids)


def _dot_t(x, w):
    return lax.dot_general(x.astype(jnp.bfloat16), w.astype(jnp.bfloat16),
                           (((1,), (1,)), ((), ())),
                           preferred_element_type=jnp.float32)


def _layer_body(x_ref, wv_ref, wo_ref, wg_ref, wu_ref, wd_ref, out_ref, a_ref):
    i = pl.program_id(0)
    t = pl.program_id(1)
    ts = t * TB

    @pl.when(i == 0)
    def _compute_attn():
        x = x_ref[pl.ds(ts, TB), :]
        v = _dot_t(x, wv_ref[...])
        ri = lax.broadcasted_iota(jnp.int32, (KD, H), 0)
        ci = lax.broadcasted_iota(jnp.int32, (KD, H), 1)
        p = ((ci % HD == ri % HD) & (ci // (G * HD) == ri // HD))
        vt = lax.dot_general(v.astype(jnp.bfloat16), p.astype(jnp.bfloat16),
                             (((1,), (0,)), ((), ())),
                             preferred_element_type=jnp.float32)
        a_ref[pl.ds(ts, TB), :] = _dot_t(vt, wo_ref[...])

    a = a_ref[pl.ds(ts, TB), :]
    gate = _dot_t(a, wg_ref[...])
    up = _dot_t(a, wu_ref[...])
    m = gate * jax.nn.sigmoid(gate) * up
    contrib = _dot_t(m, wd_ref[...])

    @pl.when(i == 0)
    def _init():
        out_ref[pl.ds(ts, TB), :] = contrib

    @pl.when(i > 0)
    def _acc():
        out_ref[pl.ds(ts, TB), :] = out_ref[pl.ds(ts, TB), :] + contrib


def _layer(x, wv, wo, wg, wu, wd):
    return pl.pallas_call(
        _layer_body,
        grid=(NI, NT),
        in_specs=[
            pl.BlockSpec((S, H), lambda i, t: (0, 0)),
            pl.BlockSpec((KD, H), lambda i, t: (0, 0)),
            pl.BlockSpec((H, H), lambda i, t: (0, 0)),
            pl.BlockSpec((IB, H), lambda i, t: (i, 0)),
            pl.BlockSpec((IB, H), lambda i, t: (i, 0)),
            pl.BlockSpec((H, IB), lambda i, t: (0, i)),
        ],
        out_specs=pl.BlockSpec((S, H), lambda i, t: (0, 0)),
        out_shape=jax.ShapeDtypeStruct((S, H), jnp.float32),
        scratch_shapes=[pltpu.VMEM((S, H), jnp.float32)],
        compiler_params=pltpu.CompilerParams(
            dimension_semantics=("arbitrary", "arbitrary"),
            vmem_limit_bytes=100 * 1024 * 1024,
        ),
    )(x, wv, wo, wg, wu, wd)


def _topk_body(h_ref, w_ref, vals_ref, idx_ref):
    logits = _dot_t(h_ref[...], w_ref[...])
    cols = lax.broadcasted_iota(jnp.int32, (TBO, V), 1)
    l = logits
    vals = []
    idxs = []
    for _ in range(TOPK):
        m = jnp.max(l, axis=1, keepdims=True)
        am = jnp.min(jnp.where(l == m, cols, V), axis=1, keepdims=True)
        vals.append(m)
        idxs.append(am)
        l = jnp.where(cols == am, -jnp.inf, l)
    vals_ref[...] = jnp.concatenate(vals, axis=1)
    idx_ref[...] = jnp.concatenate(idxs, axis=1)


def _logits_topk(h, w_out):
    return pl.pallas_call(
        _topk_body,
        grid=(NTO,),
        in_specs=[
            pl.BlockSpec((TBO, H), lambda t: (t, 0)),
            pl.BlockSpec((V, H), lambda t: (0, 0)),
        ],
        out_specs=[
            pl.BlockSpec((TBO, TOPK), lambda t: (t, 0)),
            pl.BlockSpec((TBO, TOPK), lambda t: (t, 0)),
        ],
        out_shape=[
            jax.ShapeDtypeStruct((S, TOPK), jnp.float32),
            jax.ShapeDtypeStruct((S, TOPK), jnp.int32),
        ],
        compiler_params=pltpu.CompilerParams(
            dimension_semantics=("arbitrary",),
            vmem_limit_bytes=100 * 1024 * 1024,
        ),
    )(h, w_out)


def kernel(input_ids, top_k, embed_table, Wq, Wk, Wv, Wo, Wg, Wu, Wd, W_out):
    del top_k, Wq, Wk
    ids = input_ids.reshape(S).astype(jnp.int32)
    h = _sc_gather(embed_table, ids)
    for layer in range(L):
        h = _layer(h, Wv[layer], Wo[layer], Wg[layer], Wu[layer], Wd[layer])
    vals, idx = _logits_topk(h, W_out)
    return vals.reshape(1, S, TOPK), idx.reshape(1, S, TOPK)
